# baseline (speedup 1.0000x reference)
import functools

import jax
import jax.numpy as jnp
from jax import lax
from jax.experimental import pallas as pl
from jax.experimental.pallas import tpu as pltpu
from jax.experimental.pallas import tpu_sc as plsc

VOCAB = 100000
EMBED_DIM = 64
BATCH = 4096
HIST = 50
N = BATCH * HIST

NUM_CORES = 2
NUM_SUBCORES = 16
NW = NUM_CORES * NUM_SUBCORES
PER_W = N // NW
CHUNK = 128
HALF = CHUNK // 2
NCHUNK = PER_W // CHUNK
LANES = 16

_mesh = plsc.VectorSubcoreMesh(core_axis_name="c", subcore_axis_name="s")


@functools.partial(
    pl.kernel,
    mesh=_mesh,
    out_type=jax.ShapeDtypeStruct((N // 2, 2 * EMBED_DIM), jnp.float32),
    compiler_params=pltpu.CompilerParams(use_tc_tiling_on_sc=False),
    scratch_types=[
        pltpu.VMEM((NCHUNK, CHUNK), jnp.int32),
        pltpu.VMEM((2, HALF, EMBED_DIM), jnp.float32),
        pltpu.VMEM((2, HALF, EMBED_DIM), jnp.float32),
        pltpu.SemaphoreType.DMA,
        pltpu.SemaphoreType.DMA,
        pltpu.SemaphoreType.DMA,
        pltpu.SemaphoreType.DMA,
    ],
)
def _emb_lookup(x_hbm, table_hbm, out_hbm, idx_v, rows_e, rows_o,
                gsem0, gsem1, wsem0, wsem1):
    wid = lax.axis_index("s") * NUM_CORES + lax.axis_index("c")
    base = wid * (PER_W // 2)

    pltpu.sync_copy(x_hbm.at[wid], idx_v)

    gsems = (gsem0, gsem1)
    wsems = (wsem0, wsem1)

    def gather(j, b):
        pltpu.async_copy(
            table_hbm.at[idx_v.at[j, pl.ds(0, HALF)]], rows_e.at[b],
            gsems[b])
        pltpu.async_copy(
            table_hbm.at[idx_v.at[j, pl.ds(HALF, HALF)]], rows_o.at[b],
            gsems[b])

    def wait_gather(b):
        pltpu.make_async_copy(
            table_hbm.at[idx_v.at[0, pl.ds(0, HALF)]], rows_e.at[b],
            gsems[b]).wait()
        pltpu.make_async_copy(
            table_hbm.at[idx_v.at[0, pl.ds(0, HALF)]], rows_o.at[b],
            gsems[b]).wait()

    def write_out(j, b):
        pltpu.async_copy(
            rows_e.at[b],
            out_hbm.at[pl.ds(base + j * HALF, HALF), pl.ds(0, EMBED_DIM)],
            wsems[b])
        pltpu.async_copy(
            rows_o.at[b],
            out_hbm.at[pl.ds(base + j * HALF, HALF),
                       pl.ds(EMBED_DIM, EMBED_DIM)],
            wsems[b])

    def wait_write(b):
        pltpu.make_async_copy(
            rows_e.at[b],
            out_hbm.at[pl.ds(base, HALF), pl.ds(0, EMBED_DIM)],
            wsems[b]).wait()
        pltpu.make_async_copy(
            rows_o.at[b],
            out_hbm.at[pl.ds(base, HALF), pl.ds(0, EMBED_DIM)],
            wsems[b]).wait()

    gather(0, 0)
    gather(1, 1)

    def chunk_body(j, _):
        for b in range(2):
            @pl.when(j % 2 == b)
            def _():
                wait_gather(b)
                write_out(j, b)

        @pl.when(j + 2 < NCHUNK)
        def _():
            for b in range(2):
                @pl.when(j % 2 == b)
                def _():
                    wait_write(b)
                    gather(j + 2, b)
        return 0

    lax.fori_loop(0, NCHUNK, chunk_body, 0)

    for b in range(2):
        wait_write(b)


def kernel(x, table):
    xw = x.reshape(NW, NCHUNK, HALF, 2).astype(jnp.int32)
    xr = jnp.concatenate([xw[..., 0], xw[..., 1]], axis=-1)
    out = _emb_lookup(xr, table)
    return out.reshape(BATCH, HIST, EMBED_DIM)


# speedup vs baseline: 1.5560x; 1.5560x over previous
import functools

import jax
import jax.numpy as jnp
from jax import lax
from jax.experimental import pallas as pl
from jax.experimental.pallas import tpu as pltpu
from jax.experimental.pallas import tpu_sc as plsc

VOCAB = 100000
EMBED_DIM = 64
BATCH = 4096
HIST = 50
N = BATCH * HIST

NUM_CORES = 2
NUM_SUBCORES = 16
NW = NUM_CORES * NUM_SUBCORES
PER_W = N // NW
CHUNK = 128
NCHUNK = PER_W // CHUNK

_mesh = plsc.VectorSubcoreMesh(core_axis_name="c", subcore_axis_name="s")


@functools.partial(
    pl.kernel,
    mesh=_mesh,
    out_type=jax.ShapeDtypeStruct((N, EMBED_DIM), jnp.float32),
    compiler_params=pltpu.CompilerParams(use_tc_tiling_on_sc=False),
    scratch_types=[
        pltpu.VMEM((PER_W,), jnp.int32),
        pltpu.VMEM((2, CHUNK, EMBED_DIM), jnp.float32),
        pltpu.SemaphoreType.DMA,
        pltpu.SemaphoreType.DMA,
        pltpu.SemaphoreType.DMA,
        pltpu.SemaphoreType.DMA,
    ],
)
def _emb_lookup(x_hbm, table_hbm, out_hbm, idx_v, rows_v, gsem0, gsem1,
                wsem0, wsem1):
    wid = lax.axis_index("s") * NUM_CORES + lax.axis_index("c")
    base = wid * PER_W

    pltpu.sync_copy(x_hbm.at[pl.ds(base, PER_W)], idx_v)

    gsems = (gsem0, gsem1)
    wsems = (wsem0, wsem1)

    def gather(j, b):
        pltpu.async_copy(
            table_hbm.at[idx_v.at[pl.ds(j * CHUNK, CHUNK)]], rows_v.at[b],
            gsems[b])

    gather(0, 0)
    gather(1, 1)

    def chunk_body(j, _):
        for b in range(2):
            @pl.when(j % 2 == b)
            def _():
                pltpu.make_async_copy(
                    table_hbm.at[idx_v.at[pl.ds(0, CHUNK)]], rows_v.at[b],
                    gsems[b]
                ).wait()
                pltpu.async_copy(
                    rows_v.at[b],
                    out_hbm.at[pl.ds(base + j * CHUNK, CHUNK)],
                    wsems[b],
                )

        @pl.when(j + 2 < NCHUNK)
        def _():
            for b in range(2):
                @pl.when(j % 2 == b)
                def _():
                    pltpu.make_async_copy(
                        rows_v.at[b],
                        out_hbm.at[pl.ds(base, CHUNK)],
                        wsems[b],
                    ).wait()
                    gather(j + 2, b)
        return 0

    lax.fori_loop(0, NCHUNK, chunk_body, 0)

    for b in range(2):
        pltpu.make_async_copy(
            rows_v.at[b], out_hbm.at[pl.ds(base, CHUNK)], wsems[b]
        ).wait()


def kernel(x, table):
    out = _emb_lookup(x.reshape(N).astype(jnp.int32), table)
    return out.reshape(BATCH, HIST, EMBED_DIM)


# speedup vs baseline: 1.6025x; 1.0299x over previous
import functools

import jax
import jax.numpy as jnp
from jax import lax
from jax.experimental import pallas as pl
from jax.experimental.pallas import tpu as pltpu
from jax.experimental.pallas import tpu_sc as plsc

VOCAB = 100000
EMBED_DIM = 64
BATCH = 4096
HIST = 50
N = BATCH * HIST

NUM_CORES = 2
NUM_SUBCORES = 16
NW = NUM_CORES * NUM_SUBCORES
PER_W = N // NW
CHUNK = 128
NCHUNK = PER_W // CHUNK

_mesh = plsc.VectorSubcoreMesh(core_axis_name="c", subcore_axis_name="s")


@functools.partial(
    pl.kernel,
    mesh=_mesh,
    out_type=jax.ShapeDtypeStruct((N, EMBED_DIM), jnp.float32),
    compiler_params=pltpu.CompilerParams(use_tc_tiling_on_sc=False),
    scratch_types=[
        pltpu.VMEM((PER_W,), jnp.int32),
        pltpu.VMEM((4, CHUNK, EMBED_DIM), jnp.float32),
        pltpu.SemaphoreType.DMA,
        pltpu.SemaphoreType.DMA,
        pltpu.SemaphoreType.DMA,
        pltpu.SemaphoreType.DMA,
        pltpu.SemaphoreType.DMA,
        pltpu.SemaphoreType.DMA,
        pltpu.SemaphoreType.DMA,
        pltpu.SemaphoreType.DMA,
    ],
)
def _emb_lookup(x_hbm, table_hbm, out_hbm, idx_v, rows_v, gsem0, gsem1,
                gsem2, gsem3, wsem0, wsem1, wsem2, wsem3):
    wid = lax.axis_index("s") * NUM_CORES + lax.axis_index("c")
    base = wid * PER_W

    pltpu.sync_copy(x_hbm.at[pl.ds(base, PER_W)], idx_v)

    gsems = (gsem0, gsem1, gsem2, gsem3)
    wsems = (wsem0, wsem1, wsem2, wsem3)
    NBUF = 4

    def gather(j, b):
        pltpu.async_copy(
            table_hbm.at[idx_v.at[pl.ds(j * CHUNK, CHUNK)]], rows_v.at[b],
            gsems[b])

    for b in range(4):
        gather(b, b)

    def chunk_body(j, _):
        for b in range(NBUF):
            @pl.when(j % NBUF == b)
            def _():
                pltpu.make_async_copy(
                    table_hbm.at[idx_v.at[pl.ds(0, CHUNK)]], rows_v.at[b],
                    gsems[b]
                ).wait()
                pltpu.async_copy(
                    rows_v.at[b],
                    out_hbm.at[pl.ds(base + j * CHUNK, CHUNK)],
                    wsems[b],
                )

        @pl.when(j + NBUF < NCHUNK)
        def _():
            for b in range(NBUF):
                @pl.when(j % NBUF == b)
                def _():
                    pltpu.make_async_copy(
                        rows_v.at[b],
                        out_hbm.at[pl.ds(base, CHUNK)],
                        wsems[b],
                    ).wait()
                    gather(j + NBUF, b)
        return 0

    lax.fori_loop(0, NCHUNK, chunk_body, 0)

    for b in range(4):
        pltpu.make_async_copy(
            rows_v.at[b], out_hbm.at[pl.ds(base, CHUNK)], wsems[b]
        ).wait()


def kernel(x, table):
    out = _emb_lookup(x.reshape(N).astype(jnp.int32), table)
    return out.reshape(BATCH, HIST, EMBED_DIM)


# speedup vs baseline: 1.6853x; 1.0517x over previous
import functools

import jax
import jax.numpy as jnp
from jax import lax
from jax.experimental import pallas as pl
from jax.experimental.pallas import tpu as pltpu
from jax.experimental.pallas import tpu_sc as plsc

VOCAB = 100000
EMBED_DIM = 64
BATCH = 4096
HIST = 50
N = BATCH * HIST

NUM_CORES = 2
NUM_SUBCORES = 16
NW = NUM_CORES * NUM_SUBCORES
B_PER_W = BATCH // NW
CHUNK = B_PER_W
NCHUNK = HIST
NBUF = 4

_mesh = plsc.VectorSubcoreMesh(core_axis_name="c", subcore_axis_name="s")


@functools.partial(
    pl.kernel,
    mesh=_mesh,
    out_type=jax.ShapeDtypeStruct((HIST, NW, B_PER_W, EMBED_DIM),
                                  jnp.float32),
    compiler_params=pltpu.CompilerParams(use_tc_tiling_on_sc=False),
    scratch_types=[
        pltpu.VMEM((NCHUNK, CHUNK), jnp.int32),
        pltpu.VMEM((NBUF, CHUNK, EMBED_DIM), jnp.float32),
        pltpu.SemaphoreType.DMA,
        pltpu.SemaphoreType.DMA,
        pltpu.SemaphoreType.DMA,
        pltpu.SemaphoreType.DMA,
        pltpu.SemaphoreType.DMA,
        pltpu.SemaphoreType.DMA,
        pltpu.SemaphoreType.DMA,
        pltpu.SemaphoreType.DMA,
    ],
)
def _emb_lookup(x_hbm, table_hbm, out_hbm, idx_v, rows_v, gsem0, gsem1,
                gsem2, gsem3, wsem0, wsem1, wsem2, wsem3):
    wid = lax.axis_index("s") * NUM_CORES + lax.axis_index("c")

    pltpu.sync_copy(x_hbm.at[:, wid], idx_v)

    gsems = (gsem0, gsem1, gsem2, gsem3)
    wsems = (wsem0, wsem1, wsem2, wsem3)

    def gather(j, b):
        pltpu.async_copy(table_hbm.at[idx_v.at[j]], rows_v.at[b], gsems[b])

    for b in range(NBUF):
        gather(b, b)

    def chunk_body(j, _):
        for b in range(NBUF):
            @pl.when(j % NBUF == b)
            def _():
                pltpu.make_async_copy(
                    table_hbm.at[idx_v.at[0]], rows_v.at[b], gsems[b]
                ).wait()
                pltpu.async_copy(
                    rows_v.at[b], out_hbm.at[j, wid], wsems[b])

        @pl.when(j + NBUF < NCHUNK)
        def _():
            for b in range(NBUF):
                @pl.when(j % NBUF == b)
                def _():
                    pltpu.make_async_copy(
                        rows_v.at[b], out_hbm.at[0, wid], wsems[b]
                    ).wait()
                    gather(j + NBUF, b)
        return 0

    lax.fori_loop(0, NCHUNK, chunk_body, 0)

    for b in range(NBUF):
        pltpu.make_async_copy(
            rows_v.at[b], out_hbm.at[0, wid], wsems[b]
        ).wait()


def kernel(x, table):
    xt = x.T.astype(jnp.int32).reshape(HIST, NW, B_PER_W)
    out = _emb_lookup(xt, table)
    return out.reshape(HIST, BATCH, EMBED_DIM).transpose(1, 0, 2)


# speedup vs baseline: 1.6918x; 1.0039x over previous
import functools

import jax
import jax.numpy as jnp
from jax import lax
from jax.experimental import pallas as pl
from jax.experimental.pallas import tpu as pltpu
from jax.experimental.pallas import tpu_sc as plsc

VOCAB = 100000
EMBED_DIM = 64
BATCH = 4096
HIST = 50
N = BATCH * HIST

NUM_CORES = 2
NUM_SUBCORES = 16
NW = NUM_CORES * NUM_SUBCORES
B_PER_W = BATCH // NW
CHUNK = B_PER_W
NCHUNK = HIST
NBUF = 8

_mesh = plsc.VectorSubcoreMesh(core_axis_name="c", subcore_axis_name="s")


@functools.partial(
    pl.kernel,
    mesh=_mesh,
    out_type=jax.ShapeDtypeStruct((HIST, NW, B_PER_W, EMBED_DIM),
                                  jnp.float32),
    compiler_params=pltpu.CompilerParams(use_tc_tiling_on_sc=False),
    scratch_types=[
        pltpu.VMEM((NCHUNK, CHUNK), jnp.int32),
        pltpu.VMEM((NBUF, CHUNK, EMBED_DIM), jnp.float32),
    ] + [pltpu.SemaphoreType.DMA] * 16,
)
def _emb_lookup(x_hbm, table_hbm, out_hbm, idx_v, rows_v, *sems):
    wid = lax.axis_index("s") * NUM_CORES + lax.axis_index("c")

    pltpu.sync_copy(x_hbm.at[:, wid], idx_v)

    gsems = sems[:NBUF]
    wsems = sems[NBUF:]

    def gather(j, b):
        pltpu.async_copy(table_hbm.at[idx_v.at[j]], rows_v.at[b], gsems[b])

    for b in range(NBUF):
        gather(b, b)

    def chunk_body(j, _):
        for b in range(NBUF):
            @pl.when(j % NBUF == b)
            def _():
                pltpu.make_async_copy(
                    table_hbm.at[idx_v.at[0]], rows_v.at[b], gsems[b]
                ).wait()
                pltpu.async_copy(
                    rows_v.at[b], out_hbm.at[j, wid], wsems[b])

        @pl.when(j + NBUF < NCHUNK)
        def _():
            for b in range(NBUF):
                @pl.when(j % NBUF == b)
                def _():
                    pltpu.make_async_copy(
                        rows_v.at[b], out_hbm.at[0, wid], wsems[b]
                    ).wait()
                    gather(j + NBUF, b)
        return 0

    lax.fori_loop(0, NCHUNK, chunk_body, 0)

    for b in range(NBUF):
        pltpu.make_async_copy(
            rows_v.at[b], out_hbm.at[0, wid], wsems[b]
        ).wait()


def kernel(x, table):
    xt = x.T.astype(jnp.int32).reshape(HIST, NW, B_PER_W)
    out = _emb_lookup(xt, table)
    return out.reshape(HIST, BATCH, EMBED_DIM).transpose(1, 0, 2)
